# baseline (device time: 130418 ns/iter reference)
import jax
import jax.numpy as jnp
from jax import lax
from jax.experimental import pallas as pl
from jax.experimental.pallas import tpu as pltpu

T = 1024
D = 2048
V_SHARD = 16384
NX, NY, NZ = 2, 2, 4
N_DEV = NX * NY * NZ
N_REP = NX * NZ
BN = V_SHARD // N_REP


def kernel(x, W, labels):
    labels2d = labels.reshape(T, 1)

    def body(x_any, w_any, lab_ref, out_ref,
             xv_ref, wv_ref, recv_ref, stats_ref,
             xcopy_sem, wcopy_sem, send_sems, recv_sems):
        my_x = lax.axis_index("x")
        my_y = lax.axis_index("y")
        my_z = lax.axis_index("z")
        my_id = my_x * (NY * NZ) + my_y * NZ + my_z
        r = my_x * NZ + my_z
        col0 = r * BN

        recv_ref[...] = jnp.zeros((N_DEV, T, 2), jnp.float32)

        xcopy = pltpu.make_async_copy(x_any, xv_ref, xcopy_sem)
        xcopy.start()
        wcopy = pltpu.make_async_copy(
            w_any.at[:, pl.ds(col0, BN)], wv_ref, wcopy_sem)
        wcopy.start()

        barrier = pltpu.get_barrier_semaphore()
        for p in range(N_DEV):
            px, py, pz = p // (NY * NZ), (p // NZ) % NY, p % NZ

            @pl.when(p != my_id)
            def _():
                pl.semaphore_signal(
                    barrier, inc=1,
                    device_id=(px, py, pz),
                    device_id_type=pl.DeviceIdType.MESH,
                )
        pl.semaphore_wait(barrier, N_DEV - 1)
        xcopy.wait()
        wcopy.wait()

        logits = jnp.dot(xv_ref[...], wv_ref[...],
                         preferred_element_type=jnp.float32)
        s_part = jnp.sum(jnp.exp(logits), axis=1, keepdims=True)
        local_label = lab_ref[...] - (my_y * V_SHARD + col0)
        col_ids = lax.broadcasted_iota(jnp.int32, (T, BN), 1)
        hit = col_ids == local_label
        g_part = jnp.sum(jnp.where(hit, logits, 0.0), axis=1,
                         keepdims=True)
        stats_ref[:, 0:1] = s_part
        stats_ref[:, 1:2] = g_part

        send_rdmas = []
        for p in range(N_DEV):
            px, py, pz = p // (NY * NZ), (p // NZ) % NY, p % NZ
            rdma = pltpu.make_async_remote_copy(
                src_ref=stats_ref,
                dst_ref=recv_ref.at[my_id],
                send_sem=send_sems.at[p],
                recv_sem=recv_sems.at[my_id],
                device_id=(px, py, pz),
                device_id_type=pl.DeviceIdType.MESH,
            )
            send_rdmas.append(rdma)

            @pl.when(p != my_id)
            def _():
                rdma.start()

        for p in range(N_DEV):
            recv_d = pltpu.make_async_remote_copy(
                src_ref=stats_ref,
                dst_ref=recv_ref.at[p],
                send_sem=send_sems.at[p],
                recv_sem=recv_sems.at[p],
                device_id=(0, 0, 0),
                device_id_type=pl.DeviceIdType.MESH,
            )

            @pl.when(p != my_id)
            def _():
                recv_d.wait_recv()
                send_rdmas[p].wait_send()

        total = jnp.sum(recv_ref[...], axis=0) + stats_ref[...]
        out_ref[...] = jnp.log(total[:, 0:1]) - total[:, 1:2]

    out = pl.pallas_call(
        body,
        in_specs=[
            pl.BlockSpec(memory_space=pl.ANY),
            pl.BlockSpec(memory_space=pl.ANY),
            pl.BlockSpec(memory_space=pltpu.VMEM),
        ],
        out_specs=pl.BlockSpec(memory_space=pltpu.VMEM),
        out_shape=jax.ShapeDtypeStruct((T, 1), jnp.float32),
        scratch_shapes=[
            pltpu.VMEM((T, D), jnp.float32),
            pltpu.VMEM((D, BN), jnp.float32),
            pltpu.VMEM((N_DEV, T, 2), jnp.float32),
            pltpu.VMEM((T, 2), jnp.float32),
            pltpu.SemaphoreType.DMA,
            pltpu.SemaphoreType.DMA,
            pltpu.SemaphoreType.DMA((N_DEV,)),
            pltpu.SemaphoreType.DMA((N_DEV,)),
        ],
        compiler_params=pltpu.CompilerParams(
            collective_id=0,
            vmem_limit_bytes=64 * 1024 * 1024,
        ),
    )(x, W, labels2d)
    return out.reshape(T)


# device time: 68337 ns/iter; 1.9085x vs baseline; 1.9085x over previous
import jax
import jax.numpy as jnp
from jax import lax
from jax.experimental import pallas as pl
from jax.experimental.pallas import tpu as pltpu

T = 1024
D = 2048
V_SHARD = 16384
HALF = V_SHARD // 2
BN = 2048
NC = HALF // BN
N_PLANE = 4


def kernel(x, W, labels):
    labels2d = labels.reshape(T, 1)

    def body(x_any, w_any, lab_ref, out_ref,
             xv_ref, wv_ref, recv_ref, stats_ref,
             xcopy_sem, wcopy_sems, send_sems, recv_sems):
        my_x = lax.axis_index("x")
        my_y = lax.axis_index("y")
        my_z = lax.axis_index("z")
        my_slot = my_x * 2 + my_y
        col0 = my_x * HALF

        recv_ref[...] = jnp.zeros((N_PLANE, T, 2), jnp.float32)

        xcopy = pltpu.make_async_copy(x_any, xv_ref, xcopy_sem)
        xcopy.start()
        wcopies = [
            pltpu.make_async_copy(
                w_any.at[:, pl.ds(col0 + j * BN, BN)],
                wv_ref.at[j % 2],
                wcopy_sems.at[j % 2],
            )
            for j in range(NC)
        ]
        wcopies[0].start()

        barrier = pltpu.get_barrier_semaphore()
        for p in range(N_PLANE):
            px, py = p // 2, p % 2

            @pl.when(p != my_slot)
            def _():
                pl.semaphore_signal(
                    barrier, inc=1,
                    device_id=(px, py, my_z),
                    device_id_type=pl.DeviceIdType.MESH,
                )
        pl.semaphore_wait(barrier, N_PLANE - 1)

        wcopies[1].start()
        xcopy.wait()

        base = my_y * V_SHARD + col0
        col_iota = lax.broadcasted_iota(jnp.int32, (T, BN), 1)
        s_acc = jnp.zeros((T, 1), jnp.float32)
        g_acc = jnp.zeros((T, 1), jnp.float32)
        for j in range(NC):
            wcopies[j].wait()
            logits = jnp.dot(xv_ref[...], wv_ref[j % 2],
                             preferred_element_type=jnp.float32)
            if j + 2 < NC:
                wcopies[j + 2].start()
            s_acc += jnp.sum(jnp.exp(logits), axis=1, keepdims=True)
            hit = col_iota == (lab_ref[...] - (base + j * BN))
            g_acc += jnp.sum(jnp.where(hit, logits, 0.0), axis=1,
                             keepdims=True)
        stats_ref[:, 0:1] = s_acc
        stats_ref[:, 1:2] = g_acc

        send_rdmas = []
        for p in range(N_PLANE):
            px, py = p // 2, p % 2
            rdma = pltpu.make_async_remote_copy(
                src_ref=stats_ref,
                dst_ref=recv_ref.at[my_slot],
                send_sem=send_sems.at[p],
                recv_sem=recv_sems.at[my_slot],
                device_id=(px, py, my_z),
                device_id_type=pl.DeviceIdType.MESH,
            )
            send_rdmas.append(rdma)

            @pl.when(p != my_slot)
            def _():
                rdma.start()

        for p in range(N_PLANE):
            recv_d = pltpu.make_async_remote_copy(
                src_ref=stats_ref,
                dst_ref=recv_ref.at[p],
                send_sem=send_sems.at[p],
                recv_sem=recv_sems.at[p],
                device_id=(0, 0, 0),
                device_id_type=pl.DeviceIdType.MESH,
            )

            @pl.when(p != my_slot)
            def _():
                recv_d.wait_recv()
                send_rdmas[p].wait_send()

        total = jnp.sum(recv_ref[...], axis=0) + stats_ref[...]
        out_ref[...] = jnp.log(total[:, 0:1]) - total[:, 1:2]

    out = pl.pallas_call(
        body,
        in_specs=[
            pl.BlockSpec(memory_space=pl.ANY),
            pl.BlockSpec(memory_space=pl.ANY),
            pl.BlockSpec(memory_space=pltpu.VMEM),
        ],
        out_specs=pl.BlockSpec(memory_space=pltpu.VMEM),
        out_shape=jax.ShapeDtypeStruct((T, 1), jnp.float32),
        scratch_shapes=[
            pltpu.VMEM((T, D), jnp.float32),
            pltpu.VMEM((2, D, BN), jnp.float32),
            pltpu.VMEM((N_PLANE, T, 2), jnp.float32),
            pltpu.VMEM((T, 2), jnp.float32),
            pltpu.SemaphoreType.DMA,
            pltpu.SemaphoreType.DMA((2,)),
            pltpu.SemaphoreType.DMA((N_PLANE,)),
            pltpu.SemaphoreType.DMA((N_PLANE,)),
        ],
        compiler_params=pltpu.CompilerParams(
            collective_id=0,
            vmem_limit_bytes=64 * 1024 * 1024,
        ),
    )(x, W, labels2d)
    return out.reshape(T)


# device time: 67361 ns/iter; 1.9361x vs baseline; 1.0145x over previous
import jax
import jax.numpy as jnp
from jax import lax
from jax.experimental import pallas as pl
from jax.experimental.pallas import tpu as pltpu

T = 1024
D = 2048
V_SHARD = 16384
HALF = V_SHARD // 2
BN = 1024
NC = HALF // BN
N_PLANE = 4


def kernel(x, W, labels):
    labels2d = labels.reshape(T, 1)

    def body(x_any, w_any, lab_ref, out_ref,
             xv_ref, wv_ref, recv_ref, stats_ref,
             xcopy_sem, wcopy_sems, send_sems, recv_sems):
        my_x = lax.axis_index("x")
        my_y = lax.axis_index("y")
        my_z = lax.axis_index("z")
        my_slot = my_x * 2 + my_y
        col0 = my_x * HALF

        recv_ref[...] = jnp.zeros((N_PLANE, T, 2), jnp.float32)

        xcopy = pltpu.make_async_copy(x_any, xv_ref, xcopy_sem)
        xcopy.start()
        wcopies = [
            pltpu.make_async_copy(
                w_any.at[:, pl.ds(col0 + j * BN, BN)],
                wv_ref.at[j % 2],
                wcopy_sems.at[j % 2],
            )
            for j in range(NC)
        ]
        wcopies[0].start()
        wcopies[1].start()

        barrier = pltpu.get_barrier_semaphore()
        for p in range(N_PLANE):
            px, py = p // 2, p % 2

            @pl.when(p != my_slot)
            def _():
                pl.semaphore_signal(
                    barrier, inc=1,
                    device_id=(px, py, my_z),
                    device_id_type=pl.DeviceIdType.MESH,
                )
        pl.semaphore_wait(barrier, N_PLANE - 1)
        xcopy.wait()

        base = my_y * V_SHARD + col0
        col_iota = lax.broadcasted_iota(jnp.int32, (T, BN), 1)
        s_acc = jnp.zeros((T, 1), jnp.float32)
        g_acc = jnp.zeros((T, 1), jnp.float32)
        for j in range(NC):
            wcopies[j].wait()
            logits = jnp.dot(xv_ref[...], wv_ref[j % 2],
                             preferred_element_type=jnp.float32)
            if j + 2 < NC:
                wcopies[j + 2].start()
            s_acc += jnp.sum(jnp.exp(logits), axis=1, keepdims=True)
            hit = col_iota == (lab_ref[...] - (base + j * BN))
            g_acc += jnp.sum(jnp.where(hit, logits, 0.0), axis=1,
                             keepdims=True)
        stats_ref[:, 0:1] = s_acc
        stats_ref[:, 1:2] = g_acc

        send_rdmas = []
        for p in range(N_PLANE):
            px, py = p // 2, p % 2
            rdma = pltpu.make_async_remote_copy(
                src_ref=stats_ref,
                dst_ref=recv_ref.at[my_slot],
                send_sem=send_sems.at[p],
                recv_sem=recv_sems.at[my_slot],
                device_id=(px, py, my_z),
                device_id_type=pl.DeviceIdType.MESH,
            )
            send_rdmas.append(rdma)

            @pl.when(p != my_slot)
            def _():
                rdma.start()

        for p in range(N_PLANE):
            recv_d = pltpu.make_async_remote_copy(
                src_ref=stats_ref,
                dst_ref=recv_ref.at[p],
                send_sem=send_sems.at[p],
                recv_sem=recv_sems.at[p],
                device_id=(0, 0, 0),
                device_id_type=pl.DeviceIdType.MESH,
            )

            @pl.when(p != my_slot)
            def _():
                recv_d.wait_recv()
                send_rdmas[p].wait_send()

        total = jnp.sum(recv_ref[...], axis=0) + stats_ref[...]
        out_ref[...] = jnp.log(total[:, 0:1]) - total[:, 1:2]

    out = pl.pallas_call(
        body,
        in_specs=[
            pl.BlockSpec(memory_space=pl.ANY),
            pl.BlockSpec(memory_space=pl.ANY),
            pl.BlockSpec(memory_space=pltpu.VMEM),
        ],
        out_specs=pl.BlockSpec(memory_space=pltpu.VMEM),
        out_shape=jax.ShapeDtypeStruct((T, 1), jnp.float32),
        scratch_shapes=[
            pltpu.VMEM((T, D), jnp.float32),
            pltpu.VMEM((2, D, BN), jnp.float32),
            pltpu.VMEM((N_PLANE, T, 2), jnp.float32),
            pltpu.VMEM((T, 2), jnp.float32),
            pltpu.SemaphoreType.DMA,
            pltpu.SemaphoreType.DMA((2,)),
            pltpu.SemaphoreType.DMA((N_PLANE,)),
            pltpu.SemaphoreType.DMA((N_PLANE,)),
        ],
        compiler_params=pltpu.CompilerParams(
            collective_id=0,
            vmem_limit_bytes=64 * 1024 * 1024,
        ),
    )(x, W, labels2d)
    return out.reshape(T)
